# baseline (speedup 1.0000x reference)
import functools

import jax
import jax.numpy as jnp
from jax import lax
from jax.experimental import pallas as pl
from jax.experimental.pallas import tpu as pltpu
from jax.experimental.pallas import tpu_sc as plsc

_B = 16384
_D = 128
_NC, _NS = 2, 16
_NW = _NC * _NS
_CHUNK = 128
_CPW = _B // _NW
_NCH = _CPW // _CHUNK


@functools.cache
def _make_sc_gather():
    mesh = plsc.VectorSubcoreMesh(core_axis_name="c", subcore_axis_name="s",
                                  num_cores=_NC, num_subcores=_NS)

    @functools.partial(
        pl.kernel,
        out_type=(
            jax.ShapeDtypeStruct((_B, _D), jnp.float32),
            jax.ShapeDtypeStruct((_B, _D), jnp.float32),
        ),
        mesh=mesh,
        scratch_types=[
            pltpu.VMEM((_NCH, _CHUNK), jnp.int32),
            pltpu.VMEM((_NCH, _CHUNK), jnp.int32),
            pltpu.VMEM((2, _CHUNK, _D), jnp.float32),
            pltpu.VMEM((2, _CHUNK, _D), jnp.float32),
            pltpu.SemaphoreType.DMA,
            pltpu.SemaphoreType.DMA,
            pltpu.SemaphoreType.DMA,
            pltpu.SemaphoreType.DMA,
        ],
    )
    def _sc_gather(uidx_hbm, ridx_hbm, utab_hbm, rtab_hbm, uout_hbm,
                   rout_hbm, uidx_v, ridx_v, urows_v, rrows_v,
                   ugsem, rgsem, uwsem, rwsem):
        wid = lax.axis_index("s") * _NC + lax.axis_index("c")
        base = wid * _CPW
        pltpu.sync_copy(uidx_hbm.at[wid], uidx_v)
        pltpu.sync_copy(ridx_hbm.at[wid], ridx_v)
        ug = [None] * _NCH
        rg = [None] * _NCH
        uw = [None] * _NCH
        rw = [None] * _NCH
        for j in range(2):
            ug[j] = pltpu.async_copy(utab_hbm.at[uidx_v.at[j]],
                                     urows_v.at[j % 2], ugsem)
            rg[j] = pltpu.async_copy(rtab_hbm.at[ridx_v.at[j]],
                                     rrows_v.at[j % 2], rgsem)
        for j in range(_NCH):
            row = pl.ds(base + j * _CHUNK, _CHUNK)
            ug[j].wait()
            uw[j] = pltpu.async_copy(urows_v.at[j % 2], uout_hbm.at[row],
                                     uwsem)
            rg[j].wait()
            rw[j] = pltpu.async_copy(rrows_v.at[j % 2], rout_hbm.at[row],
                                     rwsem)
            if j + 2 < _NCH:
                uw[j].wait()
                ug[j + 2] = pltpu.async_copy(utab_hbm.at[uidx_v.at[j + 2]],
                                             urows_v.at[j % 2], ugsem)
                rw[j].wait()
                rg[j + 2] = pltpu.async_copy(rtab_hbm.at[ridx_v.at[j + 2]],
                                             rrows_v.at[j % 2], rgsem)
        uw[_NCH - 2].wait()
        rw[_NCH - 2].wait()
        uw[_NCH - 1].wait()
        rw[_NCH - 1].wait()

    return _sc_gather


_BB = 8192


def _mlp_body(r_ref, u_ref, w1_ref, b1_ref, w2_ref, b2_ref, w3t_ref, b3_ref,
              o_ref):
    w1 = w1_ref[...]
    h = jnp.dot(r_ref[...], w1[:_D], preferred_element_type=jnp.float32)
    h = h + jnp.dot(u_ref[...], w1[_D:], preferred_element_type=jnp.float32)
    h = jnp.maximum(h + b1_ref[...], 0.0)
    h = jnp.dot(h, w2_ref[...], preferred_element_type=jnp.float32)
    h = jnp.maximum(h + b2_ref[...], 0.0)
    o = jnp.dot(w3t_ref[...], h.T, preferred_element_type=jnp.float32)
    o_ref[...] = (o + b3_ref[...])[None]


def _mlp(r_emb, u_emb, W1, b1, W2, b2, W3t, b3):
    return pl.pallas_call(
        _mlp_body,
        grid=(_B // _BB,),
        in_specs=[
            pl.BlockSpec((_BB, _D), lambda i: (i, 0)),
            pl.BlockSpec((_BB, _D), lambda i: (i, 0)),
            pl.BlockSpec((2 * _D, 100), lambda i: (0, 0)),
            pl.BlockSpec((1, 100), lambda i: (0, 0)),
            pl.BlockSpec((100, 50), lambda i: (0, 0)),
            pl.BlockSpec((1, 50), lambda i: (0, 0)),
            pl.BlockSpec((1, 50), lambda i: (0, 0)),
            pl.BlockSpec((1, 1), lambda i: (0, 0)),
        ],
        out_specs=pl.BlockSpec((1, 1, _BB), lambda i: (i, 0, 0)),
        out_shape=jax.ShapeDtypeStruct((_B // _BB, 1, _BB), jnp.float32),
    )(r_emb, u_emb, W1, b1, W2, b2, W3t, b3)


def kernel(user, recipe, user_table, recipe_table, W1, b1, W2, b2, W3, b3):
    uidx = user.astype(jnp.int32).reshape(_NW, _NCH, _CHUNK)
    ridx = recipe.astype(jnp.int32).reshape(_NW, _NCH, _CHUNK)
    u_emb, r_emb = _make_sc_gather()(uidx, ridx, user_table, recipe_table)
    out = _mlp(r_emb, u_emb, W1, b1.reshape(1, -1), W2, b2.reshape(1, -1),
               W3.reshape(1, -1), b3.reshape(1, 1))
    return out.reshape(_B)


# speedup vs baseline: 1.0057x; 1.0057x over previous
import functools

import jax
import jax.numpy as jnp
from jax import lax
from jax.experimental import pallas as pl
from jax.experimental.pallas import tpu as pltpu
from jax.experimental.pallas import tpu_sc as plsc

_B = 16384
_D = 128
_NC, _NS = 2, 16
_NW = _NC * _NS
_CHUNK = 128
_CPW = _B // _NW
_NCH = _CPW // _CHUNK


@functools.cache
def _make_sc_gather():
    mesh = plsc.VectorSubcoreMesh(core_axis_name="c", subcore_axis_name="s",
                                  num_cores=_NC, num_subcores=_NS)

    @functools.partial(
        pl.kernel,
        out_type=(
            jax.ShapeDtypeStruct((_B, _D), jnp.float32),
            jax.ShapeDtypeStruct((_B, _D), jnp.float32),
        ),
        mesh=mesh,
        scratch_types=[
            pltpu.VMEM((_NCH, _CHUNK), jnp.int32),
            pltpu.VMEM((_NCH, _CHUNK), jnp.int32),
            pltpu.VMEM((2, _CHUNK, _D), jnp.float32),
            pltpu.VMEM((2, _CHUNK, _D), jnp.float32),
            pltpu.SemaphoreType.DMA,
            pltpu.SemaphoreType.DMA,
            pltpu.SemaphoreType.DMA,
            pltpu.SemaphoreType.DMA,
        ],
    )
    def _sc_gather(uidx_hbm, ridx_hbm, utab_hbm, rtab_hbm, uout_hbm,
                   rout_hbm, uidx_v, ridx_v, urows_v, rrows_v,
                   ugsem, rgsem, uwsem, rwsem):
        wid = lax.axis_index("s") * _NC + lax.axis_index("c")
        base = wid * _CPW
        pltpu.sync_copy(uidx_hbm.at[wid], uidx_v)
        pltpu.sync_copy(ridx_hbm.at[wid], ridx_v)
        ug = [None] * _NCH
        rg = [None] * _NCH
        uw = [None] * _NCH
        rw = [None] * _NCH
        for j in range(2):
            ug[j] = pltpu.async_copy(utab_hbm.at[uidx_v.at[j]],
                                     urows_v.at[j % 2], ugsem)
            rg[j] = pltpu.async_copy(rtab_hbm.at[ridx_v.at[j]],
                                     rrows_v.at[j % 2], rgsem)
        for j in range(_NCH):
            row = pl.ds(base + j * _CHUNK, _CHUNK)
            ug[j].wait()
            uw[j] = pltpu.async_copy(urows_v.at[j % 2], uout_hbm.at[row],
                                     uwsem)
            rg[j].wait()
            rw[j] = pltpu.async_copy(rrows_v.at[j % 2], rout_hbm.at[row],
                                     rwsem)
            if j + 2 < _NCH:
                uw[j].wait()
                ug[j + 2] = pltpu.async_copy(utab_hbm.at[uidx_v.at[j + 2]],
                                             urows_v.at[j % 2], ugsem)
                rw[j].wait()
                rg[j + 2] = pltpu.async_copy(rtab_hbm.at[ridx_v.at[j + 2]],
                                             rrows_v.at[j % 2], rgsem)
        uw[_NCH - 2].wait()
        rw[_NCH - 2].wait()
        uw[_NCH - 1].wait()
        rw[_NCH - 1].wait()

    return _sc_gather


_BB = 4096


def _mlp_body(r_ref, u_ref, w1_ref, b1_ref, w2_ref, b2_ref, w3t_ref, b3_ref,
              o_ref):
    w1 = w1_ref[...]
    h = jnp.dot(r_ref[...], w1[:_D], preferred_element_type=jnp.float32)
    h = h + jnp.dot(u_ref[...], w1[_D:], preferred_element_type=jnp.float32)
    h = jnp.maximum(h + b1_ref[...], 0.0)
    h = jnp.dot(h, w2_ref[...], preferred_element_type=jnp.float32)
    h = jnp.maximum(h + b2_ref[...], 0.0)
    o = jnp.dot(w3t_ref[...], h.T, preferred_element_type=jnp.float32)
    o_ref[...] = (o + b3_ref[...])[None]


def _mlp(r_emb, u_emb, W1, b1, W2, b2, W3t, b3):
    return pl.pallas_call(
        _mlp_body,
        grid=(_B // _BB,),
        in_specs=[
            pl.BlockSpec((_BB, _D), lambda i: (i, 0)),
            pl.BlockSpec((_BB, _D), lambda i: (i, 0)),
            pl.BlockSpec((2 * _D, 100), lambda i: (0, 0)),
            pl.BlockSpec((1, 100), lambda i: (0, 0)),
            pl.BlockSpec((100, 50), lambda i: (0, 0)),
            pl.BlockSpec((1, 50), lambda i: (0, 0)),
            pl.BlockSpec((1, 50), lambda i: (0, 0)),
            pl.BlockSpec((1, 1), lambda i: (0, 0)),
        ],
        out_specs=pl.BlockSpec((1, 1, _BB), lambda i: (i, 0, 0)),
        out_shape=jax.ShapeDtypeStruct((_B // _BB, 1, _BB), jnp.float32),
    )(r_emb, u_emb, W1, b1, W2, b2, W3t, b3)


def kernel(user, recipe, user_table, recipe_table, W1, b1, W2, b2, W3, b3):
    uidx = user.astype(jnp.int32).reshape(_NW, _NCH, _CHUNK)
    ridx = recipe.astype(jnp.int32).reshape(_NW, _NCH, _CHUNK)
    u_emb, r_emb = _make_sc_gather()(uidx, ridx, user_table, recipe_table)
    out = _mlp(r_emb, u_emb, W1, b1.reshape(1, -1), W2, b2.reshape(1, -1),
               W3.reshape(1, -1), b3.reshape(1, 1))
    return out.reshape(_B)
